# Initial kernel scaffold; baseline (speedup 1.0000x reference)
import functools

import jax
import jax.numpy as jnp
from jax import lax
from jax.experimental import pallas as pl
from jax.experimental.pallas import tpu as pltpu
from jax.experimental.pallas import tpu_sc as plsc

N = 10000
D = 128
E = 320000
NC = 2
NS = 16
NW = NC * NS
EB = 128
NB = -(-E // (NW * EB))
E_PAD = NW * NB * EB
N_PAD = 10240
INIT_ROWS = N_PAD // NS
OUT_ROWS = N // NS
BN_EPS = 1e-5


def _sc_segment_sum(h, src_r, dst_r, zeros):
    mesh = plsc.VectorSubcoreMesh(
        core_axis_name="c", subcore_axis_name="s",
        num_cores=NC, num_subcores=NS)

    @functools.partial(
        pl.kernel,
        out_type=jax.ShapeDtypeStruct((NC, N, D), jnp.float32),
        mesh=mesh,
        scratch_types=[
            pltpu.VMEM((NB, EB), jnp.int32),
            pltpu.VMEM((NB, EB), jnp.int32),
            pltpu.VMEM((EB, D), jnp.float32),
            pltpu.VMEM_SHARED((N_PAD, D), jnp.float32),
            pltpu.SemaphoreType.DMA,
        ],
    )
    def seg_sum(h_hbm, src_hbm, dst_hbm, z_hbm, out_hbm,
                src_v, dst_v, rows_v, acc, sem):
        c = lax.axis_index("c")
        s = lax.axis_index("s")
        wid = c * NS + s
        pltpu.sync_copy(z_hbm.at[pl.ds(s * INIT_ROWS, INIT_ROWS)],
                        acc.at[pl.ds(s * INIT_ROWS, INIT_ROWS)])
        pltpu.sync_copy(src_hbm.at[wid], src_v)
        pltpu.sync_copy(dst_hbm.at[wid], dst_v)
        plsc.subcore_barrier()

        def body(j, carry):
            pltpu.async_copy(h_hbm.at[src_v.at[j]], rows_v, sem).wait()
            pltpu.sync_copy(rows_v, acc.at[dst_v.at[j]], add=True)
            return carry

        lax.fori_loop(0, NB, body, 0)
        plsc.subcore_barrier()
        pltpu.sync_copy(acc.at[pl.ds(s * OUT_ROWS, OUT_ROWS)],
                        out_hbm.at[c, pl.ds(s * OUT_ROWS, OUT_ROWS)])

    return seg_sum(h, src_r, dst_r, zeros)


def _tc_tail(h, parts, snorm, W1, b1, W2, b2, gamma, beta):
    def body(h_ref, p_ref, sn_ref, w1_ref, b1_ref, w2_ref, b2_ref,
             g_ref, be_ref, o_ref):
        hv = h_ref[...]
        hh = hv + p_ref[0] + p_ref[1]
        a = jnp.dot(hh, w1_ref[...], preferred_element_type=jnp.float32,
                    precision=lax.Precision.HIGHEST) + b1_ref[...]
        a = jnp.maximum(a, 0.0)
        z = jnp.dot(a, w2_ref[...], preferred_element_type=jnp.float32,
                    precision=lax.Precision.HIGHEST) + b2_ref[...]
        z = z * sn_ref[...]
        mean = jnp.mean(z, axis=0, keepdims=True)
        zc = z - mean
        var = jnp.mean(zc * zc, axis=0, keepdims=True)
        zn = zc * lax.rsqrt(var + BN_EPS) * g_ref[...] + be_ref[...]
        o_ref[...] = hv + jnp.maximum(zn, 0.0)

    return pl.pallas_call(
        body,
        out_shape=jax.ShapeDtypeStruct((N, D), jnp.float32),
    )(h, parts, snorm, W1, b1.reshape(1, D), W2, b2.reshape(1, D),
      gamma.reshape(1, D), beta.reshape(1, D))


def kernel(h, edge_index, snorm_n, W1, b1, W2, b2, gamma, beta):
    src = edge_index[0]
    dst = edge_index[1]
    pad = E_PAD - E
    src_p = jnp.concatenate([src, jnp.zeros((pad,), jnp.int32)])
    dst_p = jnp.concatenate(
        [dst, N + (jnp.arange(pad, dtype=jnp.int32) % (N_PAD - N))])
    src_r = src_p.reshape(NW, NB, EB)
    dst_r = dst_p.reshape(NW, NB, EB)
    zeros = jnp.zeros((N_PAD, D), jnp.float32)
    parts = _sc_segment_sum(h, src_r, dst_r, zeros)
    return _tc_tail(h, parts, snorm_n, W1, b1, W2, b2, gamma, beta)


# speedup vs baseline: 4.9053x; 4.9053x over previous
import functools

import jax
import jax.numpy as jnp
from jax import lax
from jax.experimental import pallas as pl
from jax.experimental.pallas import tpu as pltpu
from jax.experimental.pallas import tpu_sc as plsc

N = 10000
D = 128
E = 320000
NC = 2
NS = 16
NW = NC * NS
EB = 128
NB = -(-E // (NW * EB))
E_PAD = NW * NB * EB
N_PAD = 10240
INIT_ROWS = N_PAD // NS
OUT_ROWS = N_PAD // NS
BN_EPS = 1e-5


def _sc_segment_sum(h, src_r, dst_r, zeros):
    mesh = plsc.VectorSubcoreMesh(
        core_axis_name="c", subcore_axis_name="s",
        num_cores=NC, num_subcores=NS)

    @functools.partial(
        pl.kernel,
        out_type=jax.ShapeDtypeStruct((NC, N_PAD, D), jnp.float32),
        mesh=mesh,
        scratch_types=[
            pltpu.VMEM((NB, EB), jnp.int32),
            pltpu.VMEM((NB, EB), jnp.int32),
            pltpu.VMEM((EB, D), jnp.float32),
            pltpu.VMEM_SHARED((N_PAD, D), jnp.float32),
            pltpu.SemaphoreType.DMA,
        ],
    )
    def seg_sum(h_hbm, src_hbm, dst_hbm, z_hbm, out_hbm,
                src_v, dst_v, rows_v, acc, sem):
        c = lax.axis_index("c")
        s = lax.axis_index("s")
        wid = c * NS + s
        pltpu.sync_copy(z_hbm.at[pl.ds(s * INIT_ROWS, INIT_ROWS)],
                        acc.at[pl.ds(s * INIT_ROWS, INIT_ROWS)])
        pltpu.sync_copy(src_hbm.at[wid], src_v)
        pltpu.sync_copy(dst_hbm.at[wid], dst_v)
        plsc.subcore_barrier()

        def body(j, carry):
            pltpu.async_copy(h_hbm.at[src_v.at[j]], rows_v, sem).wait()
            pltpu.sync_copy(rows_v, acc.at[dst_v.at[j]], add=True)
            return carry

        lax.fori_loop(0, NB, body, 0)
        plsc.subcore_barrier()
        pltpu.sync_copy(acc.at[pl.ds(s * OUT_ROWS, OUT_ROWS)],
                        out_hbm.at[c, pl.ds(s * OUT_ROWS, OUT_ROWS)])

    return seg_sum(h, src_r, dst_r, zeros)


def _tc_tail(h, parts, snorm, W1, b1, W2, b2, gamma, beta):
    def body(h_ref, p_ref, sn_ref, w1_ref, b1_ref, w2_ref, b2_ref,
             g_ref, be_ref, o_ref):
        hv = h_ref[...]
        hh = hv + p_ref[0, :N] + p_ref[1, :N]
        a = jnp.dot(hh, w1_ref[...], preferred_element_type=jnp.float32,
                    precision=lax.Precision.HIGHEST) + b1_ref[...]
        a = jnp.maximum(a, 0.0)
        z = jnp.dot(a, w2_ref[...], preferred_element_type=jnp.float32,
                    precision=lax.Precision.HIGHEST) + b2_ref[...]
        z = z * sn_ref[...]
        mean = jnp.mean(z, axis=0, keepdims=True)
        zc = z - mean
        var = jnp.mean(zc * zc, axis=0, keepdims=True)
        zn = zc * lax.rsqrt(var + BN_EPS) * g_ref[...] + be_ref[...]
        o_ref[...] = hv + jnp.maximum(zn, 0.0)

    return pl.pallas_call(
        body,
        out_shape=jax.ShapeDtypeStruct((N, D), jnp.float32),
    )(h, parts, snorm, W1, b1.reshape(1, D), W2, b2.reshape(1, D),
      gamma.reshape(1, D), beta.reshape(1, D))


def kernel(h, edge_index, snorm_n, W1, b1, W2, b2, gamma, beta):
    src = edge_index[0]
    dst = edge_index[1]
    pad = E_PAD - E
    src_p = jnp.concatenate([src, jnp.zeros((pad,), jnp.int32)])
    dst_p = jnp.concatenate(
        [dst, N + (jnp.arange(pad, dtype=jnp.int32) % (N_PAD - N))])
    src_r = src_p.reshape(NW, NB, EB)
    dst_r = dst_p.reshape(NW, NB, EB)
    zeros = jnp.zeros((N_PAD, D), jnp.float32)
    parts = _sc_segment_sum(h, src_r, dst_r, zeros)
    return _tc_tail(h, parts, snorm_n, W1, b1, W2, b2, gamma, beta)
